# baseline (speedup 1.0000x reference)
import jax
import jax.numpy as jnp
from jax import lax
from jax.experimental import pallas as pl
from jax.experimental.pallas import tpu as pltpu
from jax.experimental.pallas import tpu_sc as plsc

BATCH = 4096
LENGTH = 200
HIDDEN = 64
VOCAB = 1000000

NUM_CORES = 2
NUM_SUBCORES = 16
NW = NUM_CORES * NUM_SUBCORES

N = BATCH * LENGTH
POS = 512
UNITS = N // POS
U_PER_W = UNITS // NW
UPR = BATCH // POS
GCHUNK = 128
NG = POS // GCHUNK


def _emb_body(wt_hbm, idx_hbm, out_hbm, idx_v, g_v, t_v, sem_g, sem_w):
    cid = lax.axis_index("c")
    sid = lax.axis_index("s")
    wid = cid * NUM_SUBCORES + sid

    def stage(uu, slot):
        ug = wid * U_PER_W + uu
        pltpu.sync_copy(idx_hbm.at[pl.ds(ug * POS, POS)], idx_v.at[slot])
        for j in range(NG):
            pltpu.async_copy(
                wt_hbm.at[idx_v.at[slot, pl.ds(j * GCHUNK, GCHUNK)]],
                g_v.at[slot, pl.ds(j * GCHUNK, GCHUNK)], sem_g)

    stage(0, 0)

    def unit(u, carry):
        par = lax.rem(u, 2)
        ug = wid * U_PER_W + u
        l = ug // UPR
        b0 = lax.rem(ug, UPR) * POS

        for j in range(NG):
            pltpu.make_async_copy(
                wt_hbm.at[idx_v.at[par, pl.ds(j * GCHUNK, GCHUNK)]],
                g_v.at[par, pl.ds(j * GCHUNK, GCHUNK)], sem_g).wait()

        @pl.when(u + 1 < U_PER_W)
        def _():
            stage(u + 1, 1 - par)

        @pl.when(u >= 1)
        def _():
            ug1 = ug - 1
            l1 = ug1 // UPR
            b1 = lax.rem(ug1, UPR) * POS
            pltpu.make_async_copy(
                t_v, out_hbm.at[l1, :, pl.ds(b1, POS)],
                sem_w).wait()

        it16 = lax.iota(jnp.int32, 16)

        def tp(b, c2):
            bb = jnp.full((16,), b, jnp.int32)
            for c16 in range(HIDDEN // 16):
                v = g_v[par, b, pl.ds(c16 * 16, 16)]
                plsc.store_scatter(t_v, [c16 * 16 + it16, bb], v)
            return c2
        lax.fori_loop(0, POS, tp, 0)

        pltpu.async_copy(t_v, out_hbm.at[l, :, pl.ds(b0, POS)],
                         sem_w)
        return carry

    lax.fori_loop(0, U_PER_W, unit, 0)

    ugt = wid * U_PER_W + U_PER_W - 1
    lt = ugt // UPR
    bt = lax.rem(ugt, UPR) * POS
    pltpu.make_async_copy(
        t_v, out_hbm.at[lt, :, pl.ds(bt, POS)], sem_w).wait()


@jax.jit
def _emb(wt, idxf):
    mesh = plsc.VectorSubcoreMesh(core_axis_name="c", subcore_axis_name="s")
    k = pl.kernel(
        _emb_body,
        out_type=jax.ShapeDtypeStruct((LENGTH, HIDDEN, BATCH), jnp.float32),
        mesh=mesh,
        scratch_types=[
            pltpu.VMEM((2, POS), jnp.int32),
            pltpu.VMEM((2, POS, HIDDEN), jnp.float32),
            pltpu.VMEM((HIDDEN, POS), jnp.float32),
            pltpu.SemaphoreType.DMA,
            pltpu.SemaphoreType.DMA,
        ],
        compiler_params=pltpu.CompilerParams(use_tc_tiling_on_sc=False,
                                              needs_layout_passes=False),
    )
    return k(wt, idxf)


def kernel(inputs, weight):
    idxf = inputs.T.reshape(-1)
    out = _emb(weight, idxf.astype(jnp.int32))
    return out.transpose(2, 0, 1)


# speedup vs baseline: 1.4880x; 1.4880x over previous
import jax
import jax.numpy as jnp
from jax import lax
from jax.experimental import pallas as pl
from jax.experimental.pallas import tpu as pltpu
from jax.experimental.pallas import tpu_sc as plsc

BATCH = 4096
LENGTH = 200
HIDDEN = 64
VOCAB = 1000000

NUM_CORES = 2
NUM_SUBCORES = 16
NW = NUM_CORES * NUM_SUBCORES

N = BATCH * LENGTH
POS = 512
UNITS = N // POS
U_PER_W = UNITS // NW
GCHUNK = 128
NG = POS // GCHUNK


def _emb_body(wt_hbm, idx_hbm, out_hbm, idx_v, g_v, sem_g, sem_w):
    cid = lax.axis_index("c")
    sid = lax.axis_index("s")
    wid = cid * NUM_SUBCORES + sid

    def stage(uu, slot):
        ug = wid * U_PER_W + uu
        pltpu.sync_copy(idx_hbm.at[pl.ds(ug * POS, POS)], idx_v.at[slot])
        for j in range(NG):
            pltpu.async_copy(
                wt_hbm.at[idx_v.at[slot, pl.ds(j * GCHUNK, GCHUNK)]],
                g_v.at[slot, pl.ds(j * GCHUNK, GCHUNK)], sem_g)

    stage(0, 0)

    def unit(u, carry):
        par = lax.rem(u, 2)
        ug = wid * U_PER_W + u

        for j in range(NG):
            pltpu.make_async_copy(
                wt_hbm.at[idx_v.at[par, pl.ds(j * GCHUNK, GCHUNK)]],
                g_v.at[par, pl.ds(j * GCHUNK, GCHUNK)], sem_g).wait()

        @pl.when(u >= 1)
        def _():
            ug1 = ug - 1
            pltpu.make_async_copy(
                g_v.at[1 - par],
                out_hbm.at[pl.ds(ug1 * POS, POS)], sem_w).wait()

        @pl.when(u + 1 < U_PER_W)
        def _():
            stage(u + 1, 1 - par)

        pltpu.async_copy(g_v.at[par],
                         out_hbm.at[pl.ds(ug * POS, POS)], sem_w)
        return carry

    lax.fori_loop(0, U_PER_W, unit, 0)

    ugt = wid * U_PER_W + U_PER_W - 1
    part = (U_PER_W - 1) % 2
    pltpu.make_async_copy(
        g_v.at[part], out_hbm.at[pl.ds(ugt * POS, POS)], sem_w).wait()


@jax.jit
def _emb(wt, idxf):
    mesh = plsc.VectorSubcoreMesh(core_axis_name="c", subcore_axis_name="s")
    k = pl.kernel(
        _emb_body,
        out_type=jax.ShapeDtypeStruct((N, HIDDEN), jnp.float32),
        mesh=mesh,
        scratch_types=[
            pltpu.VMEM((2, POS), jnp.int32),
            pltpu.VMEM((2, POS, HIDDEN), jnp.float32),
            pltpu.SemaphoreType.DMA,
            pltpu.SemaphoreType.DMA,
        ],
        compiler_params=pltpu.CompilerParams(use_tc_tiling_on_sc=False,
                                              needs_layout_passes=False),
    )
    return k(wt, idxf)


def kernel(inputs, weight):
    idxf = inputs.reshape(-1)
    out = _emb(weight, idxf.astype(jnp.int32))
    return out.reshape(BATCH, LENGTH, HIDDEN)
